# baseline (speedup 1.0000x reference)
import functools

import jax
import jax.numpy as jnp
from jax.experimental import pallas as pl
from jax.experimental.pallas import tpu as pltpu
from jax.experimental.pallas import tpu_sc as plsc

_N = 5000
_NCLS = 20
_IMG_H = 600.0
_IMG_W = 800.0
_SCORE_TH = 0.05
_NMS_TH = 0.3

_B = 128
_NPAD = 5120
_CP = 24
_RK = 8
_RKO = 16


def _nms_kernel(cc_ref, cr_ref, ac_ref, ar_ref, valid_ref, keep_ref,
                diag_ref, keeprow_ref):
    valid = valid_ref[:, :]
    keep_ref[:, :] = jnp.zeros_like(valid)
    keeprow_ref[:, :] = jnp.zeros((_NPAD, _CP), jnp.float32)
    maxv = jnp.max(jnp.sum(valid, axis=1)).astype(jnp.int32)
    nb = (maxv + (_B - 1)) // _B

    lane = jax.lax.broadcasted_iota(jnp.int32, (_CP, _B), 1)

    def process_block(J, carry):
        colbase = J * _B
        y1c = cc_ref[0, :, pl.ds(colbase, _B)][None]
        x1c = cc_ref[1, :, pl.ds(colbase, _B)][None]
        y2c = cc_ref[2, :, pl.ds(colbase, _B)][None]
        x2c = cc_ref[3, :, pl.ds(colbase, _B)][None]
        areac = ac_ref[:, pl.ds(colbase, _B)][None]

        def offdiag(I, sup):
            rowbase = I * _B
            for r in range(_B // _RKO):
                rb = rowbase + r * _RKO
                y1r = cr_ref[0, pl.ds(rb, _RKO), :][:, :, None]
                x1r = cr_ref[1, pl.ds(rb, _RKO), :][:, :, None]
                y2r = cr_ref[2, pl.ds(rb, _RKO), :][:, :, None]
                x2r = cr_ref[3, pl.ds(rb, _RKO), :][:, :, None]
                arear = ar_ref[pl.ds(rb, _RKO), :][:, :, None]
                krow = keeprow_ref[pl.ds(rb, _RKO), :][:, :, None]
                iy = jnp.clip(jnp.minimum(y2r, y2c) - jnp.maximum(y1r, y1c), 0.0)
                ix = jnp.clip(jnp.minimum(x2r, x2c) - jnp.maximum(x1r, x1c), 0.0)
                inter = iy * ix
                iou = inter / (arear + areac - inter + 1e-9)
                hit = jnp.where((iou > _NMS_TH) & (krow > 0.5), 1.0, 0.0)
                sup = jnp.maximum(sup, jnp.max(hit, axis=0))
            return sup

        sup = jax.lax.fori_loop(0, J, offdiag, jnp.zeros((_CP, _B), jnp.float32))
        keepJ = valid_ref[:, pl.ds(colbase, _B)] * (1.0 - sup)

        iota_i = jax.lax.broadcasted_iota(jnp.int32, (_RK, _CP, _B), 0)
        iota_j = jax.lax.broadcasted_iota(jnp.int32, (_RK, _CP, _B), 2)
        for r in range(_B // _RK):
            rb = colbase + r * _RK
            y1r = cr_ref[0, pl.ds(rb, _RK), :][:, :, None]
            x1r = cr_ref[1, pl.ds(rb, _RK), :][:, :, None]
            y2r = cr_ref[2, pl.ds(rb, _RK), :][:, :, None]
            x2r = cr_ref[3, pl.ds(rb, _RK), :][:, :, None]
            arear = ar_ref[pl.ds(rb, _RK), :][:, :, None]
            iy = jnp.clip(jnp.minimum(y2r, y2c) - jnp.maximum(y1r, y1c), 0.0)
            ix = jnp.clip(jnp.minimum(x2r, x2c) - jnp.maximum(x1r, x1c), 0.0)
            inter = iy * ix
            iou = inter / (arear + areac - inter + 1e-9)
            tri = iota_j > (iota_i + r * _RK)
            diag_ref[pl.ds(r * _RK, _RK), :, :] = jnp.where(
                (iou > _NMS_TH) & tri, 1.0, 0.0)

        def dslab(g, keepJ):
            base = g * _RK
            slab = diag_ref[pl.ds(base, _RK), :, :]
            for k in range(_RK):
                i = base + k
                alive = jnp.max(jnp.where(lane == i, keepJ, 0.0), axis=1,
                                keepdims=True)
                keepJ = jnp.where((slab[k] > 0.5) & (alive > 0.5), 0.0, keepJ)
            return keepJ

        keepJ = jax.lax.fori_loop(0, _B // _RK, dslab, keepJ)
        keep_ref[:, pl.ds(colbase, _B)] = keepJ
        keeprow_ref[pl.ds(colbase, _B), :] = jnp.transpose(keepJ, (1, 0))
        return carry

    jax.lax.fori_loop(0, nb, process_block, 0)


_BTOT = _NCLS * _N + 96
_NSUB = 16
_NWORK = 32
_BPW = _BTOT // _NWORK


@functools.lru_cache(maxsize=None)
def _build_sc_unsort():
    @functools.partial(
        pl.kernel,
        mesh=plsc.VectorSubcoreMesh(core_axis_name="c", subcore_axis_name="s"),
        out_type=jax.ShapeDtypeStruct((2, _BTOT), jnp.float32),
        scratch_types=[
            pltpu.VMEM((_BPW,), jnp.int32),
            pltpu.VMEM((_BPW,), jnp.float32),
            pltpu.VMEM_SHARED((_BTOT,), jnp.float32),
        ],
    )
    def sc_unsort(vals_hbm, idx_hbm, zeros_hbm, out_hbm, idx_v, vals_v, acc_sh):
        cid = jax.lax.axis_index("c")
        sid = jax.lax.axis_index("s")
        base = (cid * _NSUB + sid) * _BPW

        @pl.when(sid == 0)
        def _():
            pltpu.sync_copy(zeros_hbm, acc_sh)

        plsc.subcore_barrier()
        pltpu.sync_copy(idx_hbm.at[pl.ds(base, _BPW)], idx_v)
        pltpu.sync_copy(vals_hbm.at[pl.ds(base, _BPW)], vals_v)
        pltpu.sync_copy(vals_v, acc_sh.at[idx_v], add=True)
        plsc.subcore_barrier()

        @pl.when(sid == 0)
        def _():
            pltpu.sync_copy(acc_sh, out_hbm.at[cid])

    return sc_unsort


def _sc_unsort(vals_flat, idx_flat, zeros):
    return _build_sc_unsort()(vals_flat, idx_flat, zeros)


def _pad_nc(a):
    return jnp.pad(a, ((0, _NPAD - _N), (0, _CP - _NCLS)))


def kernel(predicted_roi_bboxes, predicted_roi_score):
    b = predicted_roi_bboxes.reshape(_N, _NCLS + 1, 4)
    by1 = jnp.clip(b[..., 0], 0.0, _IMG_H)
    bx1 = jnp.clip(b[..., 1], 0.0, _IMG_W)
    by2 = jnp.clip(b[..., 2], 0.0, _IMG_H)
    bx2 = jnp.clip(b[..., 3], 0.0, _IMG_W)
    prob = jax.nn.softmax(predicted_roi_score, axis=1)
    p = prob[:, 1:]
    y1, x1, y2, x2 = by1[:, 1:], bx1[:, 1:], by2[:, 1:], bx2[:, 1:]
    mask = p > _SCORE_TH
    s = jnp.where(mask, p, -1.0)
    order = jnp.argsort(-s, axis=0)

    sy1 = jnp.take_along_axis(y1, order, axis=0)
    sx1 = jnp.take_along_axis(x1, order, axis=0)
    sy2 = jnp.take_along_axis(y2, order, axis=0)
    sx2 = jnp.take_along_axis(x2, order, axis=0)
    sp = jnp.take_along_axis(s, order, axis=0)
    sv = (sp > _SCORE_TH).astype(jnp.float32)
    area = jnp.clip(sy2 - sy1, 0.0) * jnp.clip(sx2 - sx1, 0.0)

    coords_r = jnp.stack([_pad_nc(sy1), _pad_nc(sx1), _pad_nc(sy2), _pad_nc(sx2)])
    coords_c = jnp.transpose(coords_r, (0, 2, 1))
    ar = _pad_nc(area)
    ac = ar.T
    vc = _pad_nc(sv).T

    keep_s = pl.pallas_call(
        _nms_kernel,
        out_shape=jax.ShapeDtypeStruct((_CP, _NPAD), jnp.float32),
        scratch_shapes=[pltpu.VMEM((_B, _CP, _B), jnp.float32),
                        pltpu.VMEM((_NPAD, _CP), jnp.float32)],
    )(coords_c, coords_r, ac, ar, vc)

    keep_nc = keep_s[:_NCLS, :_N].T

    tgt = order * _NCLS + jnp.arange(_NCLS, dtype=order.dtype)[None, :]
    idx_flat = jnp.concatenate([
        tgt.reshape(-1),
        jnp.arange(_N * _NCLS, _BTOT, dtype=order.dtype),
    ]).astype(jnp.int32)
    vals_flat = jnp.pad(keep_nc.reshape(-1), (0, _BTOT - _N * _NCLS))
    parts = _sc_unsort(vals_flat, idx_flat, jnp.zeros((_BTOT,), jnp.float32))
    kf = (parts[0] + parts[1])[:_N * _NCLS].reshape(_N, _NCLS)

    boxes_out = jnp.stack([y1, x1, y2, x2], axis=-1) * kf[:, :, None]
    lbl = jnp.arange(_NCLS, dtype=jnp.float32)[None, :] * kf
    rows = jnp.concatenate([boxes_out, (p * kf)[:, :, None], lbl[:, :, None]],
                           axis=-1)
    return jnp.transpose(rows, (1, 0, 2)).reshape(_NCLS * _N, 6)
